# baseline (speedup 1.0000x reference)
import functools

import jax
import jax.numpy as jnp
from jax import lax
from jax.experimental import pallas as pl
from jax.experimental.pallas import tpu as pltpu
from jax.experimental.pallas import tpu_sc as plsc

_N = 10000
_NP = 10016
_E = 330000
_EW = 10496
_EP = _EW * 32
_NCH = _EW // 128


def _sweep_body(F, h_hbm, asrc_hbm, adst_hbm, src_hbm, dst_hbm, out_hbm,
                as_v, ad_v, src_v, dst_v, ex0_v, ex1_v, rows0_v, rows1_v,
                outb0_v, outb1_v, zbuf_v, acc_sh, g0, g1, s0, s1):
    FP = F + 16
    c = lax.axis_index("c")
    s = lax.axis_index("s")
    wid = c * 16 + s

    pltpu.sync_copy(asrc_hbm, as_v)
    pltpu.sync_copy(adst_hbm, ad_v)
    pltpu.sync_copy(src_hbm.at[pl.ds(wid * _NCH, _NCH)], src_v)
    pltpu.sync_copy(dst_hbm.at[pl.ds(wid * _NCH, _NCH)], dst_v)

    zero16 = jnp.zeros((16,), jnp.float32)
    for r in range(32):
        for k in range(FP // 16):
            zbuf_v[r, pl.ds(k * 16, 16)] = zero16
    rows_per_sub = _NP // 16

    def zeroacc(t, carry):
        pltpu.sync_copy(zbuf_v, acc_sh.at[pl.ds(s * rows_per_sub + t * 32, 32)])
        return carry

    lax.fori_loop(0, rows_per_sub // 32, zeroacc, 0)
    rem = rows_per_sub % 32
    if rem:
        pltpu.sync_copy(
            zbuf_v.at[pl.ds(0, rem)],
            acc_sh.at[pl.ds(s * rows_per_sub + rows_per_sub - rem, rem)])
    plsc.subcore_barrier()

    lane = lax.iota(jnp.int32, 16)

    def compute_ex(j, ex_v):
        for i in range(8):
            s16 = src_v[j, pl.ds(i * 16, 16)]
            d16 = dst_v[j, pl.ds(i * 16, 16)]
            a = plsc.load_gather(as_v, [s16]) + plsc.load_gather(ad_v, [d16])
            a = jnp.maximum(a, 0.2 * a)
            ex_v[pl.ds(i * 16, 16)] = jnp.exp(a)

    def scale(rows_v, ex_v, outb_v):
        def scale16(i, inner):
            ex16 = ex_v[pl.ds(i * 16, 16)]
            for l in range(16):
                e = i * 16 + l
                exs = ex16[l]
                for k in range(F // 16):
                    outb_v[e, pl.ds(k * 16, 16)] = (
                        rows_v[e, pl.ds(k * 16, 16)] * exs)
                outb_v[e, pl.ds(F, 16)] = jnp.where(lane == 0, exs, 0.0)
            return inner

        lax.fori_loop(0, 8, scale16, 0)

    def issue_gather(j, rows_v, sem):
        return pltpu.async_copy(h_hbm.at[src_v.at[j]], rows_v, sem)

    def drain_gather(rows_v, sem):
        pltpu.make_async_copy(h_hbm.at[src_v.at[0]], rows_v, sem).wait()

    def issue_scatter(outb_v, j, sem):
        pltpu.async_copy(outb_v, acc_sh.at[dst_v.at[j]], sem, add=True)

    def drain_scatter(outb_v, sem):
        pltpu.make_async_copy(outb_v, acc_sh.at[dst_v.at[0]], sem).wait()

    def do_pair(jj, first):
        j0 = jj * 2
        j1 = j0 + 1
        compute_ex(j0, ex0_v)
        gd1 = issue_gather(j1, rows1_v, g1)
        drain_gather(rows0_v, g0)
        if not first:
            drain_scatter(outb0_v, s0)
        scale(rows0_v, ex0_v, outb0_v)
        issue_scatter(outb0_v, j0, s0)
        compute_ex(j1, ex1_v)
        if first:
            issue_gather(j0 + 2, rows0_v, g0)
        else:
            @pl.when(jj < _NCH // 2 - 1)
            def _():
                issue_gather(j0 + 2, rows0_v, g0)
        gd1.wait()
        if not first:
            drain_scatter(outb1_v, s1)
        scale(rows1_v, ex1_v, outb1_v)
        issue_scatter(outb1_v, j1, s1)

    issue_gather(0, rows0_v, g0)
    do_pair(0, True)

    def pair_body(jj, carry):
        do_pair(jj, False)
        return carry

    lax.fori_loop(1, _NCH // 2, pair_body, 0)
    drain_scatter(outb0_v, s0)
    drain_scatter(outb1_v, s1)
    plsc.subcore_barrier()

    pltpu.sync_copy(acc_sh.at[pl.ds(s * rows_per_sub, rows_per_sub)],
                    out_hbm.at[c, pl.ds(s * rows_per_sub, rows_per_sub)])


@functools.lru_cache(maxsize=None)
def _make_sweep(F):
    FP = F + 16
    mesh = plsc.VectorSubcoreMesh(core_axis_name="c", subcore_axis_name="s")
    return pl.kernel(
        functools.partial(_sweep_body, F),
        out_type=jax.ShapeDtypeStruct((2, _NP, FP), jnp.float32),
        mesh=mesh,
        scratch_types=[
            pltpu.VMEM((_NP,), jnp.float32),
            pltpu.VMEM((_NP,), jnp.float32),
            pltpu.VMEM((_NCH, 128), jnp.int32),
            pltpu.VMEM((_NCH, 128), jnp.int32),
            pltpu.VMEM((128,), jnp.float32),
            pltpu.VMEM((128,), jnp.float32),
            pltpu.VMEM((128, F), jnp.float32),
            pltpu.VMEM((128, F), jnp.float32),
            pltpu.VMEM((128, FP), jnp.float32),
            pltpu.VMEM((128, FP), jnp.float32),
            pltpu.VMEM((32, FP), jnp.float32),
            pltpu.VMEM_SHARED((_NP, FP), jnp.float32),
            pltpu.SemaphoreType.DMA,
            pltpu.SemaphoreType.DMA,
            pltpu.SemaphoreType.DMA,
            pltpu.SemaphoreType.DMA,
        ],
        name=f"gat_sweep_f{F}",
        compiler_params=pltpu.CompilerParams(
            needs_layout_passes=False, use_tc_tiling_on_sc=False),
    )


_R = 2000


def _lin_alpha_body(x_ref, w_ref, as_ref, ad_ref, h_ref, asrc_ref, adst_ref):
    h = jnp.dot(x_ref[...], w_ref[...], preferred_element_type=jnp.float32)
    h_ref[...] = h
    asrc_ref[...] = jnp.dot(h, as_ref[...], preferred_element_type=jnp.float32)
    adst_ref[...] = jnp.dot(h, ad_ref[...], preferred_element_type=jnp.float32)


@functools.lru_cache(maxsize=None)
def _make_lin_alpha(Fin, F2):
    return pl.pallas_call(
        _lin_alpha_body,
        grid=(_N // _R,),
        in_specs=[
            pl.BlockSpec((_R, Fin), lambda i: (i, 0)),
            pl.BlockSpec((Fin, F2), lambda i: (0, 0)),
            pl.BlockSpec((F2, 1), lambda i: (0, 0)),
            pl.BlockSpec((F2, 1), lambda i: (0, 0)),
        ],
        out_specs=[
            pl.BlockSpec((_R, F2), lambda i: (i, 0)),
            pl.BlockSpec((_R, 1), lambda i: (i, 0)),
            pl.BlockSpec((_R, 1), lambda i: (i, 0)),
        ],
        out_shape=[
            jax.ShapeDtypeStruct((_N, F2), jnp.float32),
            jax.ShapeDtypeStruct((_N, 1), jnp.float32),
            jax.ShapeDtypeStruct((_N, 1), jnp.float32),
        ],
    )


def _finmm_body(F, a0_ref, a1_ref, bp_ref, w_ref, as_ref, ad_ref,
                h_ref, asrc_ref, adst_ref):
    accs = a0_ref[...] + a1_ref[...]
    x = jax.nn.relu(accs[:, :F] / accs[:, F:F + 1] + bp_ref[...])
    h = jnp.dot(x, w_ref[...], preferred_element_type=jnp.float32)
    h_ref[...] = h
    asrc_ref[...] = jnp.dot(h, as_ref[...], preferred_element_type=jnp.float32)
    adst_ref[...] = jnp.dot(h, ad_ref[...], preferred_element_type=jnp.float32)


@functools.lru_cache(maxsize=None)
def _make_finmm(F, F2):
    FP = F + 16
    return pl.pallas_call(
        functools.partial(_finmm_body, F),
        grid=(_N // _R,),
        in_specs=[
            pl.BlockSpec((_R, FP), lambda i: (i, 0)),
            pl.BlockSpec((_R, FP), lambda i: (i, 0)),
            pl.BlockSpec((1, F), lambda i: (0, 0)),
            pl.BlockSpec((F, F2), lambda i: (0, 0)),
            pl.BlockSpec((F2, 1), lambda i: (0, 0)),
            pl.BlockSpec((F2, 1), lambda i: (0, 0)),
        ],
        out_specs=[
            pl.BlockSpec((_R, F2), lambda i: (i, 0)),
            pl.BlockSpec((_R, 1), lambda i: (i, 0)),
            pl.BlockSpec((_R, 1), lambda i: (i, 0)),
        ],
        out_shape=[
            jax.ShapeDtypeStruct((_N, F2), jnp.float32),
            jax.ShapeDtypeStruct((_N, 1), jnp.float32),
            jax.ShapeDtypeStruct((_N, 1), jnp.float32),
        ],
    )


def _poolfc_body(a0_ref, a1_ref, b3_ref, wfe_ref, bfe_ref, z_ref, gmax_sc):
    i = pl.program_id(0)
    accs = a0_ref[...] + a1_ref[...]
    h = jax.nn.relu(accs[:, :64] / accs[:, 64:65] + b3_ref[...])
    bm = jnp.max(h, axis=0, keepdims=True)

    @pl.when(i == 0)
    def _():
        gmax_sc[...] = bm

    @pl.when(i > 0)
    def _():
        gmax_sc[...] = jnp.maximum(gmax_sc[...], bm)

    @pl.when(i == pl.num_programs(0) - 1)
    def _():
        z_ref[...] = (
            jnp.dot(gmax_sc[...], wfe_ref[...],
                    preferred_element_type=jnp.float32) + bfe_ref[...])


def _pool_fc(acc0, acc1, b3, Wfe, bfe):
    return pl.pallas_call(
        _poolfc_body,
        grid=(_N // _R,),
        in_specs=[
            pl.BlockSpec((_R, 80), lambda i: (i, 0)),
            pl.BlockSpec((_R, 80), lambda i: (i, 0)),
            pl.BlockSpec((1, 64), lambda i: (0, 0)),
            pl.BlockSpec((64, 64), lambda i: (0, 0)),
            pl.BlockSpec((1, 64), lambda i: (0, 0)),
        ],
        out_specs=pl.BlockSpec((1, 64), lambda i: (0, 0)),
        out_shape=jax.ShapeDtypeStruct((1, 64), jnp.float32),
        scratch_shapes=[pltpu.VMEM((1, 64), jnp.float32)],
    )(acc0, acc1, b3.reshape(1, 64), Wfe, bfe.reshape(1, 64))


def _final_body(l0_ref, l1_ref, h0_ref, h1_ref, b_ref, o_ref):
    lo = l0_ref[...] + l1_ref[...]
    hi = h0_ref[...] + h1_ref[...]
    num = jnp.concatenate(
        [lo[:, :64] / lo[:, 64:65], hi[:, :64] / hi[:, 64:65]], axis=1)
    o_ref[...] = jax.nn.sigmoid(num + b_ref[...])


def _final(lo0, lo1, hi0, hi1, b3d):
    return pl.pallas_call(
        _final_body,
        grid=(_N // _R,),
        in_specs=[pl.BlockSpec((_R, 80), lambda i: (i, 0))] * 4
        + [pl.BlockSpec((1, 128), lambda i: (0, 0))],
        out_specs=pl.BlockSpec((_R, 128), lambda i: (i, 0)),
        out_shape=jax.ShapeDtypeStruct((_N, 128), jnp.float32),
    )(lo0, lo1, hi0, hi1, b3d.reshape(1, 128))


def _fcd_body(z_ref, w_ref, b_ref, o_ref):
    o_ref[...] = jax.nn.relu(
        jnp.dot(z_ref[...], w_ref[...], preferred_element_type=jnp.float32)
        + b_ref[...])


def _fcd_matvec(z, Wfd, bfd):
    K, M = Wfd.shape
    BC = 12800
    z8 = jnp.concatenate([z, jnp.zeros((7, K), z.dtype)], axis=0)
    out = pl.pallas_call(
        _fcd_body,
        grid=(M // BC,),
        in_specs=[
            pl.BlockSpec((8, K), lambda i: (0, 0)),
            pl.BlockSpec((K, BC), lambda i: (0, i)),
            pl.BlockSpec((1, BC), lambda i: (0, i)),
        ],
        out_specs=pl.BlockSpec((8, BC), lambda i: (0, i)),
        out_shape=jax.ShapeDtypeStruct((8, M), jnp.float32),
    )(z8, Wfd, bfd.reshape(1, M))
    return out[:1]


def _pad_alpha(a2d):
    return jnp.pad(a2d[:, 0], (0, _NP - _N))


def kernel(x, edge_index, batch,
           W1e, as1e, ad1e, b1e, W2e, as2e, ad2e, b2e, W3e, as3e, ad3e, b3e,
           Wfe, bfe, Wfd, bfd,
           W1d, as1d, ad1d, b1d, W2d, as2d, ad2d, b2d, W3d, as3d, ad3d, b3d):
    N = x.shape[0]
    loops = jnp.arange(N, dtype=edge_index.dtype)
    src = jnp.concatenate([edge_index[0], loops])
    dst = jnp.concatenate([edge_index[1], loops])
    srcp = jnp.pad(src, (0, _EP - _E)).reshape(_EP // 128, 128)
    dstp = jnp.pad(dst, (0, _EP - _E), constant_values=_N).reshape(
        _EP // 128, 128)

    def sweep(h, asrc2d, adst2d, F):
        return _make_sweep(F)(h, _pad_alpha(asrc2d), _pad_alpha(adst2d),
                              srcp, dstp)

    h1, sa, da = _make_lin_alpha(128, 16)(x, W1e, as1e.reshape(16, 1),
                                          ad1e.reshape(16, 1))
    acc = sweep(h1, sa, da, 16)
    h2, sa, da = _make_finmm(16, 32)(acc[0, :_N], acc[1, :_N],
                                     b1e.reshape(1, 16), W2e,
                                     as2e.reshape(32, 1), ad2e.reshape(32, 1))
    acc = sweep(h2, sa, da, 32)
    h3, sa, da = _make_finmm(32, 64)(acc[0, :_N], acc[1, :_N],
                                     b2e.reshape(1, 32), W3e,
                                     as3e.reshape(64, 1), ad3e.reshape(64, 1))
    acc = sweep(h3, sa, da, 64)
    z = _pool_fc(acc[0, :_N], acc[1, :_N], b3e, Wfe, bfe)

    d = _fcd_matvec(z, Wfd, bfd).reshape(_N, 64)
    hd1, sa, da = _make_lin_alpha(64, 64)(d, W1d, as1d.reshape(64, 1),
                                          ad1d.reshape(64, 1))
    acc = sweep(hd1, sa, da, 64)
    hd2, sa, da = _make_finmm(64, 16)(acc[0, :_N], acc[1, :_N],
                                      b1d.reshape(1, 64), W2d,
                                      as2d.reshape(16, 1), ad2d.reshape(16, 1))
    acc = sweep(hd2, sa, da, 16)
    hd3, sa, da = _make_finmm(16, 128)(acc[0, :_N], acc[1, :_N],
                                       b2d.reshape(1, 16), W3d,
                                       as3d.reshape(128, 1),
                                       ad3d.reshape(128, 1))
    acc_lo = sweep(hd3[:, :64], sa, da, 64)
    acc_hi = sweep(hd3[:, 64:], sa, da, 64)
    return _final(acc_lo[0, :_N], acc_lo[1, :_N],
                  acc_hi[0, :_N], acc_hi[1, :_N], b3d)


# speedup vs baseline: 1.1594x; 1.1594x over previous
import functools

import jax
import jax.numpy as jnp
from jax import lax
from jax.experimental import pallas as pl
from jax.experimental.pallas import tpu as pltpu
from jax.experimental.pallas import tpu_sc as plsc

_N = 10000
_NP = 10016
_E = 330000
_EW = 10496
_EP = _EW * 32
_NCH = _EW // 128


def _sweep_body(F, h_hbm, asrc_hbm, adst_hbm, src_hbm, dst_hbm, out_hbm,
                as_v, ad_v, src_v, dst_v, ex0_v, ex1_v, rows0_v, rows1_v,
                outb0_v, outb1_v, zbuf_v, acc_sh, g0, g1, s0, s1):
    FP = F + 16
    c = lax.axis_index("c")
    s = lax.axis_index("s")
    wid = c * 16 + s

    pltpu.sync_copy(asrc_hbm, as_v)
    pltpu.sync_copy(adst_hbm, ad_v)
    pltpu.sync_copy(src_hbm.at[pl.ds(wid * _NCH, _NCH)], src_v)
    pltpu.sync_copy(dst_hbm.at[pl.ds(wid * _NCH, _NCH)], dst_v)

    zero16 = jnp.zeros((16,), jnp.float32)
    for r in range(32):
        for k in range(FP // 16):
            zbuf_v[r, pl.ds(k * 16, 16)] = zero16
    rows_per_sub = _NP // 16

    def zeroacc(t, carry):
        pltpu.sync_copy(zbuf_v, acc_sh.at[pl.ds(s * rows_per_sub + t * 32, 32)])
        return carry

    lax.fori_loop(0, rows_per_sub // 32, zeroacc, 0)
    rem = rows_per_sub % 32
    if rem:
        pltpu.sync_copy(
            zbuf_v.at[pl.ds(0, rem)],
            acc_sh.at[pl.ds(s * rows_per_sub + rows_per_sub - rem, rem)])
    plsc.subcore_barrier()

    lane = lax.iota(jnp.int32, 16)

    def compute_ex(j, ex_v):
        for i in range(8):
            s16 = src_v[j, pl.ds(i * 16, 16)]
            d16 = dst_v[j, pl.ds(i * 16, 16)]
            a = plsc.load_gather(as_v, [s16]) + plsc.load_gather(ad_v, [d16])
            a = jnp.maximum(a, 0.2 * a)
            ex_v[pl.ds(i * 16, 16)] = jnp.exp(a)

    def scale(rows_v, ex_v, outb_v):
        def scale16(i, inner):
            ex16 = ex_v[pl.ds(i * 16, 16)]
            exb = [ex16[l] * jnp.ones((16,), jnp.float32) for l in range(16)]
            tails = [jnp.where(lane == 0, exb[l], 0.0) for l in range(16)]
            for l in range(16):
                outb_v[i * 16 + l, pl.ds(F, 16)] = tails[l]
            for l0 in range(0, 16, 4):
                for k in range(F // 16):
                    vals = [rows_v[i * 16 + l0 + j, pl.ds(k * 16, 16)]
                            for j in range(4)]
                    prods = [vals[j] * exb[l0 + j] for j in range(4)]
                    for j in range(4):
                        outb_v[i * 16 + l0 + j, pl.ds(k * 16, 16)] = prods[j]
            return inner

        lax.fori_loop(0, 8, scale16, 0)

    def issue_gather(j, rows_v, sem):
        return pltpu.async_copy(h_hbm.at[src_v.at[j]], rows_v, sem)

    def drain_gather(rows_v, sem):
        pltpu.make_async_copy(h_hbm.at[src_v.at[0]], rows_v, sem).wait()

    def issue_scatter(outb_v, j, sem):
        pltpu.async_copy(outb_v, acc_sh.at[dst_v.at[j]], sem, add=True)

    def drain_scatter(outb_v, sem):
        pltpu.make_async_copy(outb_v, acc_sh.at[dst_v.at[0]], sem).wait()

    def do_pair(jj, first):
        j0 = jj * 2
        j1 = j0 + 1
        compute_ex(j0, ex0_v)
        gd1 = issue_gather(j1, rows1_v, g1)
        drain_gather(rows0_v, g0)
        if not first:
            drain_scatter(outb0_v, s0)
        scale(rows0_v, ex0_v, outb0_v)
        issue_scatter(outb0_v, j0, s0)
        compute_ex(j1, ex1_v)
        if first:
            issue_gather(j0 + 2, rows0_v, g0)
        else:
            @pl.when(jj < _NCH // 2 - 1)
            def _():
                issue_gather(j0 + 2, rows0_v, g0)
        gd1.wait()
        if not first:
            drain_scatter(outb1_v, s1)
        scale(rows1_v, ex1_v, outb1_v)
        issue_scatter(outb1_v, j1, s1)

    issue_gather(0, rows0_v, g0)
    do_pair(0, True)

    def pair_body(jj, carry):
        do_pair(jj, False)
        return carry

    lax.fori_loop(1, _NCH // 2, pair_body, 0)
    drain_scatter(outb0_v, s0)
    drain_scatter(outb1_v, s1)
    plsc.subcore_barrier()

    pltpu.sync_copy(acc_sh.at[pl.ds(s * rows_per_sub, rows_per_sub)],
                    out_hbm.at[c, pl.ds(s * rows_per_sub, rows_per_sub)])


@functools.lru_cache(maxsize=None)
def _make_sweep(F):
    FP = F + 16
    mesh = plsc.VectorSubcoreMesh(core_axis_name="c", subcore_axis_name="s")
    return pl.kernel(
        functools.partial(_sweep_body, F),
        out_type=jax.ShapeDtypeStruct((2, _NP, FP), jnp.float32),
        mesh=mesh,
        scratch_types=[
            pltpu.VMEM((_NP,), jnp.float32),
            pltpu.VMEM((_NP,), jnp.float32),
            pltpu.VMEM((_NCH, 128), jnp.int32),
            pltpu.VMEM((_NCH, 128), jnp.int32),
            pltpu.VMEM((128,), jnp.float32),
            pltpu.VMEM((128,), jnp.float32),
            pltpu.VMEM((128, F), jnp.float32),
            pltpu.VMEM((128, F), jnp.float32),
            pltpu.VMEM((128, FP), jnp.float32),
            pltpu.VMEM((128, FP), jnp.float32),
            pltpu.VMEM((32, FP), jnp.float32),
            pltpu.VMEM_SHARED((_NP, FP), jnp.float32),
            pltpu.SemaphoreType.DMA,
            pltpu.SemaphoreType.DMA,
            pltpu.SemaphoreType.DMA,
            pltpu.SemaphoreType.DMA,
        ],
        name=f"gat_sweep_f{F}",
        compiler_params=pltpu.CompilerParams(
            needs_layout_passes=False, use_tc_tiling_on_sc=False),
    )


_R = 2000


def _lin_alpha_body(x_ref, w_ref, as_ref, ad_ref, h_ref, asrc_ref, adst_ref):
    h = jnp.dot(x_ref[...], w_ref[...], preferred_element_type=jnp.float32)
    h_ref[...] = h
    asrc_ref[...] = jnp.dot(h, as_ref[...], preferred_element_type=jnp.float32)
    adst_ref[...] = jnp.dot(h, ad_ref[...], preferred_element_type=jnp.float32)


@functools.lru_cache(maxsize=None)
def _make_lin_alpha(Fin, F2):
    return pl.pallas_call(
        _lin_alpha_body,
        grid=(_N // _R,),
        in_specs=[
            pl.BlockSpec((_R, Fin), lambda i: (i, 0)),
            pl.BlockSpec((Fin, F2), lambda i: (0, 0)),
            pl.BlockSpec((F2, 1), lambda i: (0, 0)),
            pl.BlockSpec((F2, 1), lambda i: (0, 0)),
        ],
        out_specs=[
            pl.BlockSpec((_R, F2), lambda i: (i, 0)),
            pl.BlockSpec((_R, 1), lambda i: (i, 0)),
            pl.BlockSpec((_R, 1), lambda i: (i, 0)),
        ],
        out_shape=[
            jax.ShapeDtypeStruct((_N, F2), jnp.float32),
            jax.ShapeDtypeStruct((_N, 1), jnp.float32),
            jax.ShapeDtypeStruct((_N, 1), jnp.float32),
        ],
    )


def _finmm_body(F, a0_ref, a1_ref, bp_ref, w_ref, as_ref, ad_ref,
                h_ref, asrc_ref, adst_ref):
    accs = a0_ref[...] + a1_ref[...]
    x = jax.nn.relu(accs[:, :F] / accs[:, F:F + 1] + bp_ref[...])
    h = jnp.dot(x, w_ref[...], preferred_element_type=jnp.float32)
    h_ref[...] = h
    asrc_ref[...] = jnp.dot(h, as_ref[...], preferred_element_type=jnp.float32)
    adst_ref[...] = jnp.dot(h, ad_ref[...], preferred_element_type=jnp.float32)


@functools.lru_cache(maxsize=None)
def _make_finmm(F, F2):
    FP = F + 16
    return pl.pallas_call(
        functools.partial(_finmm_body, F),
        grid=(_N // _R,),
        in_specs=[
            pl.BlockSpec((_R, FP), lambda i: (i, 0)),
            pl.BlockSpec((_R, FP), lambda i: (i, 0)),
            pl.BlockSpec((1, F), lambda i: (0, 0)),
            pl.BlockSpec((F, F2), lambda i: (0, 0)),
            pl.BlockSpec((F2, 1), lambda i: (0, 0)),
            pl.BlockSpec((F2, 1), lambda i: (0, 0)),
        ],
        out_specs=[
            pl.BlockSpec((_R, F2), lambda i: (i, 0)),
            pl.BlockSpec((_R, 1), lambda i: (i, 0)),
            pl.BlockSpec((_R, 1), lambda i: (i, 0)),
        ],
        out_shape=[
            jax.ShapeDtypeStruct((_N, F2), jnp.float32),
            jax.ShapeDtypeStruct((_N, 1), jnp.float32),
            jax.ShapeDtypeStruct((_N, 1), jnp.float32),
        ],
    )


def _poolfc_body(a0_ref, a1_ref, b3_ref, wfe_ref, bfe_ref, z_ref, gmax_sc):
    i = pl.program_id(0)
    accs = a0_ref[...] + a1_ref[...]
    h = jax.nn.relu(accs[:, :64] / accs[:, 64:65] + b3_ref[...])
    bm = jnp.max(h, axis=0, keepdims=True)

    @pl.when(i == 0)
    def _():
        gmax_sc[...] = bm

    @pl.when(i > 0)
    def _():
        gmax_sc[...] = jnp.maximum(gmax_sc[...], bm)

    @pl.when(i == pl.num_programs(0) - 1)
    def _():
        z_ref[...] = (
            jnp.dot(gmax_sc[...], wfe_ref[...],
                    preferred_element_type=jnp.float32) + bfe_ref[...])


def _pool_fc(acc0, acc1, b3, Wfe, bfe):
    return pl.pallas_call(
        _poolfc_body,
        grid=(_N // _R,),
        in_specs=[
            pl.BlockSpec((_R, 80), lambda i: (i, 0)),
            pl.BlockSpec((_R, 80), lambda i: (i, 0)),
            pl.BlockSpec((1, 64), lambda i: (0, 0)),
            pl.BlockSpec((64, 64), lambda i: (0, 0)),
            pl.BlockSpec((1, 64), lambda i: (0, 0)),
        ],
        out_specs=pl.BlockSpec((1, 64), lambda i: (0, 0)),
        out_shape=jax.ShapeDtypeStruct((1, 64), jnp.float32),
        scratch_shapes=[pltpu.VMEM((1, 64), jnp.float32)],
    )(acc0, acc1, b3.reshape(1, 64), Wfe, bfe.reshape(1, 64))


def _final_body(l0_ref, l1_ref, h0_ref, h1_ref, b_ref, o_ref):
    lo = l0_ref[...] + l1_ref[...]
    hi = h0_ref[...] + h1_ref[...]
    num = jnp.concatenate(
        [lo[:, :64] / lo[:, 64:65], hi[:, :64] / hi[:, 64:65]], axis=1)
    o_ref[...] = jax.nn.sigmoid(num + b_ref[...])


def _final(lo0, lo1, hi0, hi1, b3d):
    return pl.pallas_call(
        _final_body,
        grid=(_N // _R,),
        in_specs=[pl.BlockSpec((_R, 80), lambda i: (i, 0))] * 4
        + [pl.BlockSpec((1, 128), lambda i: (0, 0))],
        out_specs=pl.BlockSpec((_R, 128), lambda i: (i, 0)),
        out_shape=jax.ShapeDtypeStruct((_N, 128), jnp.float32),
    )(lo0, lo1, hi0, hi1, b3d.reshape(1, 128))


def _fcd_body(z_ref, w_ref, b_ref, o_ref):
    o_ref[...] = jax.nn.relu(
        jnp.dot(z_ref[...], w_ref[...], preferred_element_type=jnp.float32)
        + b_ref[...])


def _fcd_matvec(z, Wfd, bfd):
    K, M = Wfd.shape
    BC = 12800
    z8 = jnp.concatenate([z, jnp.zeros((7, K), z.dtype)], axis=0)
    out = pl.pallas_call(
        _fcd_body,
        grid=(M // BC,),
        in_specs=[
            pl.BlockSpec((8, K), lambda i: (0, 0)),
            pl.BlockSpec((K, BC), lambda i: (0, i)),
            pl.BlockSpec((1, BC), lambda i: (0, i)),
        ],
        out_specs=pl.BlockSpec((8, BC), lambda i: (0, i)),
        out_shape=jax.ShapeDtypeStruct((8, M), jnp.float32),
    )(z8, Wfd, bfd.reshape(1, M))
    return out[:1]


def _pad_alpha(a2d):
    return jnp.pad(a2d[:, 0], (0, _NP - _N))


def kernel(x, edge_index, batch,
           W1e, as1e, ad1e, b1e, W2e, as2e, ad2e, b2e, W3e, as3e, ad3e, b3e,
           Wfe, bfe, Wfd, bfd,
           W1d, as1d, ad1d, b1d, W2d, as2d, ad2d, b2d, W3d, as3d, ad3d, b3d):
    N = x.shape[0]
    loops = jnp.arange(N, dtype=edge_index.dtype)
    src = jnp.concatenate([edge_index[0], loops])
    dst = jnp.concatenate([edge_index[1], loops])
    srcp = jnp.pad(src, (0, _EP - _E)).reshape(_EP // 128, 128)
    dstp = jnp.pad(dst, (0, _EP - _E), constant_values=_N).reshape(
        _EP // 128, 128)

    def sweep(h, asrc2d, adst2d, F):
        return _make_sweep(F)(h, _pad_alpha(asrc2d), _pad_alpha(adst2d),
                              srcp, dstp)

    h1, sa, da = _make_lin_alpha(128, 16)(x, W1e, as1e.reshape(16, 1),
                                          ad1e.reshape(16, 1))
    acc = sweep(h1, sa, da, 16)
    h2, sa, da = _make_finmm(16, 32)(acc[0, :_N], acc[1, :_N],
                                     b1e.reshape(1, 16), W2e,
                                     as2e.reshape(32, 1), ad2e.reshape(32, 1))
    acc = sweep(h2, sa, da, 32)
    h3, sa, da = _make_finmm(32, 64)(acc[0, :_N], acc[1, :_N],
                                     b2e.reshape(1, 32), W3e,
                                     as3e.reshape(64, 1), ad3e.reshape(64, 1))
    acc = sweep(h3, sa, da, 64)
    z = _pool_fc(acc[0, :_N], acc[1, :_N], b3e, Wfe, bfe)

    d = _fcd_matvec(z, Wfd, bfd).reshape(_N, 64)
    hd1, sa, da = _make_lin_alpha(64, 64)(d, W1d, as1d.reshape(64, 1),
                                          ad1d.reshape(64, 1))
    acc = sweep(hd1, sa, da, 64)
    hd2, sa, da = _make_finmm(64, 16)(acc[0, :_N], acc[1, :_N],
                                      b1d.reshape(1, 64), W2d,
                                      as2d.reshape(16, 1), ad2d.reshape(16, 1))
    acc = sweep(hd2, sa, da, 16)
    hd3, sa, da = _make_finmm(16, 128)(acc[0, :_N], acc[1, :_N],
                                       b2d.reshape(1, 16), W3d,
                                       as3d.reshape(128, 1),
                                       ad3d.reshape(128, 1))
    acc_lo = sweep(hd3[:, :64], sa, da, 64)
    acc_hi = sweep(hd3[:, 64:], sa, da, 64)
    return _final(acc_lo[0, :_N], acc_lo[1, :_N],
                  acc_hi[0, :_N], acc_hi[1, :_N], b3d)


# speedup vs baseline: 1.1753x; 1.0137x over previous
import functools

import jax
import jax.numpy as jnp
from jax import lax
from jax.experimental import pallas as pl
from jax.experimental.pallas import tpu as pltpu
from jax.experimental.pallas import tpu_sc as plsc

_N = 10000
_NP = 10016
_E = 330000
_EW = 10496
_EP = _EW * 32
_NCH = _EW // 128


def _sweep_body(F, h_hbm, asrc_hbm, adst_hbm, src_hbm, dst_hbm, out_hbm,
                as_v, ad_v, src_v, dst_v, ex0_v, ex1_v, rows0_v, rows1_v,
                outb0_v, outb1_v, zbuf_v, acc_sh, g0, g1, s0, s1):
    FP = F + 16
    c = lax.axis_index("c")
    s = lax.axis_index("s")
    wid = c * 16 + s

    pltpu.sync_copy(asrc_hbm, as_v)
    pltpu.sync_copy(adst_hbm, ad_v)
    pltpu.sync_copy(src_hbm.at[pl.ds(wid * _NCH, _NCH)], src_v)
    pltpu.sync_copy(dst_hbm.at[pl.ds(wid * _NCH, _NCH)], dst_v)

    zero16 = jnp.zeros((16,), jnp.float32)
    for r in range(32):
        for k in range(FP // 16):
            zbuf_v[r, pl.ds(k * 16, 16)] = zero16
    rows_per_sub = _NP // 16

    def zeroacc(t, carry):
        pltpu.sync_copy(zbuf_v, acc_sh.at[pl.ds(s * rows_per_sub + t * 32, 32)])
        return carry

    lax.fori_loop(0, rows_per_sub // 32, zeroacc, 0)
    rem = rows_per_sub % 32
    if rem:
        pltpu.sync_copy(
            zbuf_v.at[pl.ds(0, rem)],
            acc_sh.at[pl.ds(s * rows_per_sub + rows_per_sub - rem, rem)])
    plsc.subcore_barrier()

    lane = lax.iota(jnp.int32, 16)

    def compute_ex(j, ex_v):
        for i0 in range(0, 8, 4):
            s16s = [src_v[j, pl.ds((i0 + i) * 16, 16)] for i in range(4)]
            d16s = [dst_v[j, pl.ds((i0 + i) * 16, 16)] for i in range(4)]
            gs = [plsc.load_gather(as_v, [s16s[i]]) for i in range(4)]
            gd = [plsc.load_gather(ad_v, [d16s[i]]) for i in range(4)]
            aa = [gs[i] + gd[i] for i in range(4)]
            aa = [jnp.maximum(a, 0.2 * a) for a in aa]
            ee = [jnp.exp(a) for a in aa]
            for i in range(4):
                ex_v[pl.ds((i0 + i) * 16, 16)] = ee[i]

    def scale(rows_v, ex_v, outb_v):
        def scale16(i, inner):
            ex16 = ex_v[pl.ds(i * 16, 16)]
            exb = [ex16[l] * jnp.ones((16,), jnp.float32) for l in range(16)]
            tails = [jnp.where(lane == 0, exb[l], 0.0) for l in range(16)]
            for l in range(16):
                outb_v[i * 16 + l, pl.ds(F, 16)] = tails[l]
            for l0 in range(0, 16, 4):
                for k in range(F // 16):
                    vals = [rows_v[i * 16 + l0 + j, pl.ds(k * 16, 16)]
                            for j in range(4)]
                    prods = [vals[j] * exb[l0 + j] for j in range(4)]
                    for j in range(4):
                        outb_v[i * 16 + l0 + j, pl.ds(k * 16, 16)] = prods[j]
            return inner

        lax.fori_loop(0, 8, scale16, 0)

    def issue_gather(j, rows_v, sem):
        return pltpu.async_copy(h_hbm.at[src_v.at[j]], rows_v, sem)

    def drain_gather(rows_v, sem):
        pltpu.make_async_copy(h_hbm.at[src_v.at[0]], rows_v, sem).wait()

    def issue_scatter(outb_v, j, sem):
        pltpu.async_copy(outb_v, acc_sh.at[dst_v.at[j]], sem, add=True)

    def drain_scatter(outb_v, sem):
        pltpu.make_async_copy(outb_v, acc_sh.at[dst_v.at[0]], sem).wait()

    def do_pair(jj, first):
        j0 = jj * 2
        j1 = j0 + 1
        compute_ex(j0, ex0_v)
        gd1 = issue_gather(j1, rows1_v, g1)
        drain_gather(rows0_v, g0)
        if not first:
            drain_scatter(outb0_v, s0)
        scale(rows0_v, ex0_v, outb0_v)
        issue_scatter(outb0_v, j0, s0)
        compute_ex(j1, ex1_v)
        if first:
            issue_gather(j0 + 2, rows0_v, g0)
        else:
            @pl.when(jj < _NCH // 2 - 1)
            def _():
                issue_gather(j0 + 2, rows0_v, g0)
        gd1.wait()
        if not first:
            drain_scatter(outb1_v, s1)
        scale(rows1_v, ex1_v, outb1_v)
        issue_scatter(outb1_v, j1, s1)

    issue_gather(0, rows0_v, g0)
    do_pair(0, True)

    def pair_body(jj, carry):
        do_pair(jj, False)
        return carry

    lax.fori_loop(1, _NCH // 2, pair_body, 0)
    drain_scatter(outb0_v, s0)
    drain_scatter(outb1_v, s1)
    plsc.subcore_barrier()

    pltpu.sync_copy(acc_sh.at[pl.ds(s * rows_per_sub, rows_per_sub)],
                    out_hbm.at[c, pl.ds(s * rows_per_sub, rows_per_sub)])


@functools.lru_cache(maxsize=None)
def _make_sweep(F):
    FP = F + 16
    mesh = plsc.VectorSubcoreMesh(core_axis_name="c", subcore_axis_name="s")
    return pl.kernel(
        functools.partial(_sweep_body, F),
        out_type=jax.ShapeDtypeStruct((2, _NP, FP), jnp.float32),
        mesh=mesh,
        scratch_types=[
            pltpu.VMEM((_NP,), jnp.float32),
            pltpu.VMEM((_NP,), jnp.float32),
            pltpu.VMEM((_NCH, 128), jnp.int32),
            pltpu.VMEM((_NCH, 128), jnp.int32),
            pltpu.VMEM((128,), jnp.float32),
            pltpu.VMEM((128,), jnp.float32),
            pltpu.VMEM((128, F), jnp.float32),
            pltpu.VMEM((128, F), jnp.float32),
            pltpu.VMEM((128, FP), jnp.float32),
            pltpu.VMEM((128, FP), jnp.float32),
            pltpu.VMEM((32, FP), jnp.float32),
            pltpu.VMEM_SHARED((_NP, FP), jnp.float32),
            pltpu.SemaphoreType.DMA,
            pltpu.SemaphoreType.DMA,
            pltpu.SemaphoreType.DMA,
            pltpu.SemaphoreType.DMA,
        ],
        name=f"gat_sweep_f{F}",
        compiler_params=pltpu.CompilerParams(
            needs_layout_passes=False, use_tc_tiling_on_sc=False),
    )


_R = 2000


def _lin_alpha_body(x_ref, w_ref, as_ref, ad_ref, h_ref, asrc_ref, adst_ref):
    h = jnp.dot(x_ref[...], w_ref[...], preferred_element_type=jnp.float32)
    h_ref[...] = h
    asrc_ref[...] = jnp.dot(h, as_ref[...], preferred_element_type=jnp.float32)
    adst_ref[...] = jnp.dot(h, ad_ref[...], preferred_element_type=jnp.float32)


@functools.lru_cache(maxsize=None)
def _make_lin_alpha(Fin, F2):
    return pl.pallas_call(
        _lin_alpha_body,
        grid=(_N // _R,),
        in_specs=[
            pl.BlockSpec((_R, Fin), lambda i: (i, 0)),
            pl.BlockSpec((Fin, F2), lambda i: (0, 0)),
            pl.BlockSpec((F2, 1), lambda i: (0, 0)),
            pl.BlockSpec((F2, 1), lambda i: (0, 0)),
        ],
        out_specs=[
            pl.BlockSpec((_R, F2), lambda i: (i, 0)),
            pl.BlockSpec((_R, 1), lambda i: (i, 0)),
            pl.BlockSpec((_R, 1), lambda i: (i, 0)),
        ],
        out_shape=[
            jax.ShapeDtypeStruct((_N, F2), jnp.float32),
            jax.ShapeDtypeStruct((_N, 1), jnp.float32),
            jax.ShapeDtypeStruct((_N, 1), jnp.float32),
        ],
    )


def _finmm_body(F, a0_ref, a1_ref, bp_ref, w_ref, as_ref, ad_ref,
                h_ref, asrc_ref, adst_ref):
    accs = a0_ref[...] + a1_ref[...]
    x = jax.nn.relu(accs[:, :F] / accs[:, F:F + 1] + bp_ref[...])
    h = jnp.dot(x, w_ref[...], preferred_element_type=jnp.float32)
    h_ref[...] = h
    asrc_ref[...] = jnp.dot(h, as_ref[...], preferred_element_type=jnp.float32)
    adst_ref[...] = jnp.dot(h, ad_ref[...], preferred_element_type=jnp.float32)


@functools.lru_cache(maxsize=None)
def _make_finmm(F, F2):
    FP = F + 16
    return pl.pallas_call(
        functools.partial(_finmm_body, F),
        grid=(_N // _R,),
        in_specs=[
            pl.BlockSpec((_R, FP), lambda i: (i, 0)),
            pl.BlockSpec((_R, FP), lambda i: (i, 0)),
            pl.BlockSpec((1, F), lambda i: (0, 0)),
            pl.BlockSpec((F, F2), lambda i: (0, 0)),
            pl.BlockSpec((F2, 1), lambda i: (0, 0)),
            pl.BlockSpec((F2, 1), lambda i: (0, 0)),
        ],
        out_specs=[
            pl.BlockSpec((_R, F2), lambda i: (i, 0)),
            pl.BlockSpec((_R, 1), lambda i: (i, 0)),
            pl.BlockSpec((_R, 1), lambda i: (i, 0)),
        ],
        out_shape=[
            jax.ShapeDtypeStruct((_N, F2), jnp.float32),
            jax.ShapeDtypeStruct((_N, 1), jnp.float32),
            jax.ShapeDtypeStruct((_N, 1), jnp.float32),
        ],
    )


def _poolfc_body(a0_ref, a1_ref, b3_ref, wfe_ref, bfe_ref, z_ref, gmax_sc):
    i = pl.program_id(0)
    accs = a0_ref[...] + a1_ref[...]
    h = jax.nn.relu(accs[:, :64] / accs[:, 64:65] + b3_ref[...])
    bm = jnp.max(h, axis=0, keepdims=True)

    @pl.when(i == 0)
    def _():
        gmax_sc[...] = bm

    @pl.when(i > 0)
    def _():
        gmax_sc[...] = jnp.maximum(gmax_sc[...], bm)

    @pl.when(i == pl.num_programs(0) - 1)
    def _():
        z_ref[...] = (
            jnp.dot(gmax_sc[...], wfe_ref[...],
                    preferred_element_type=jnp.float32) + bfe_ref[...])


def _pool_fc(acc0, acc1, b3, Wfe, bfe):
    return pl.pallas_call(
        _poolfc_body,
        grid=(_N // _R,),
        in_specs=[
            pl.BlockSpec((_R, 80), lambda i: (i, 0)),
            pl.BlockSpec((_R, 80), lambda i: (i, 0)),
            pl.BlockSpec((1, 64), lambda i: (0, 0)),
            pl.BlockSpec((64, 64), lambda i: (0, 0)),
            pl.BlockSpec((1, 64), lambda i: (0, 0)),
        ],
        out_specs=pl.BlockSpec((1, 64), lambda i: (0, 0)),
        out_shape=jax.ShapeDtypeStruct((1, 64), jnp.float32),
        scratch_shapes=[pltpu.VMEM((1, 64), jnp.float32)],
    )(acc0, acc1, b3.reshape(1, 64), Wfe, bfe.reshape(1, 64))


def _final_body(l0_ref, l1_ref, h0_ref, h1_ref, b_ref, o_ref):
    lo = l0_ref[...] + l1_ref[...]
    hi = h0_ref[...] + h1_ref[...]
    num = jnp.concatenate(
        [lo[:, :64] / lo[:, 64:65], hi[:, :64] / hi[:, 64:65]], axis=1)
    o_ref[...] = jax.nn.sigmoid(num + b_ref[...])


def _final(lo0, lo1, hi0, hi1, b3d):
    return pl.pallas_call(
        _final_body,
        grid=(_N // _R,),
        in_specs=[pl.BlockSpec((_R, 80), lambda i: (i, 0))] * 4
        + [pl.BlockSpec((1, 128), lambda i: (0, 0))],
        out_specs=pl.BlockSpec((_R, 128), lambda i: (i, 0)),
        out_shape=jax.ShapeDtypeStruct((_N, 128), jnp.float32),
    )(lo0, lo1, hi0, hi1, b3d.reshape(1, 128))


def _fcd_body(z_ref, w_ref, b_ref, o_ref):
    o_ref[...] = jax.nn.relu(
        jnp.dot(z_ref[...], w_ref[...], preferred_element_type=jnp.float32)
        + b_ref[...])


def _fcd_matvec(z, Wfd, bfd):
    K, M = Wfd.shape
    BC = 12800
    z8 = jnp.concatenate([z, jnp.zeros((7, K), z.dtype)], axis=0)
    out = pl.pallas_call(
        _fcd_body,
        grid=(M // BC,),
        in_specs=[
            pl.BlockSpec((8, K), lambda i: (0, 0)),
            pl.BlockSpec((K, BC), lambda i: (0, i)),
            pl.BlockSpec((1, BC), lambda i: (0, i)),
        ],
        out_specs=pl.BlockSpec((8, BC), lambda i: (0, i)),
        out_shape=jax.ShapeDtypeStruct((8, M), jnp.float32),
    )(z8, Wfd, bfd.reshape(1, M))
    return out[:1]


def _pad_alpha(a2d):
    return jnp.pad(a2d[:, 0], (0, _NP - _N))


def kernel(x, edge_index, batch,
           W1e, as1e, ad1e, b1e, W2e, as2e, ad2e, b2e, W3e, as3e, ad3e, b3e,
           Wfe, bfe, Wfd, bfd,
           W1d, as1d, ad1d, b1d, W2d, as2d, ad2d, b2d, W3d, as3d, ad3d, b3d):
    N = x.shape[0]
    loops = jnp.arange(N, dtype=edge_index.dtype)
    src = jnp.concatenate([edge_index[0], loops])
    dst = jnp.concatenate([edge_index[1], loops])
    srcp = jnp.pad(src, (0, _EP - _E)).reshape(_EP // 128, 128)
    dstp = jnp.pad(dst, (0, _EP - _E), constant_values=_N).reshape(
        _EP // 128, 128)

    def sweep(h, asrc2d, adst2d, F):
        return _make_sweep(F)(h, _pad_alpha(asrc2d), _pad_alpha(adst2d),
                              srcp, dstp)

    h1, sa, da = _make_lin_alpha(128, 16)(x, W1e, as1e.reshape(16, 1),
                                          ad1e.reshape(16, 1))
    acc = sweep(h1, sa, da, 16)
    h2, sa, da = _make_finmm(16, 32)(acc[0, :_N], acc[1, :_N],
                                     b1e.reshape(1, 16), W2e,
                                     as2e.reshape(32, 1), ad2e.reshape(32, 1))
    acc = sweep(h2, sa, da, 32)
    h3, sa, da = _make_finmm(32, 64)(acc[0, :_N], acc[1, :_N],
                                     b2e.reshape(1, 32), W3e,
                                     as3e.reshape(64, 1), ad3e.reshape(64, 1))
    acc = sweep(h3, sa, da, 64)
    z = _pool_fc(acc[0, :_N], acc[1, :_N], b3e, Wfe, bfe)

    d = _fcd_matvec(z, Wfd, bfd).reshape(_N, 64)
    hd1, sa, da = _make_lin_alpha(64, 64)(d, W1d, as1d.reshape(64, 1),
                                          ad1d.reshape(64, 1))
    acc = sweep(hd1, sa, da, 64)
    hd2, sa, da = _make_finmm(64, 16)(acc[0, :_N], acc[1, :_N],
                                      b1d.reshape(1, 64), W2d,
                                      as2d.reshape(16, 1), ad2d.reshape(16, 1))
    acc = sweep(hd2, sa, da, 16)
    hd3, sa, da = _make_finmm(16, 128)(acc[0, :_N], acc[1, :_N],
                                       b2d.reshape(1, 16), W3d,
                                       as3d.reshape(128, 1),
                                       ad3d.reshape(128, 1))
    acc_lo = sweep(hd3[:, :64], sa, da, 64)
    acc_hi = sweep(hd3[:, 64:], sa, da, 64)
    return _final(acc_lo[0, :_N], acc_lo[1, :_N],
                  acc_hi[0, :_N], acc_hi[1, :_N], b3d)
